# baseline (speedup 1.0000x reference)
import functools

import jax
import jax.numpy as jnp
from jax import lax
from jax.experimental import pallas as pl
from jax.experimental.pallas import tpu as pltpu
from jax.experimental.pallas import tpu_sc as plsc

MAXD = 384.0
INTERVAL = 381.0 / 127.0
WD = 0.9
WL = 0.1

BH = 32

NC = 2
NS = 16
NW = NC * NS
LANES = 16


def _tc_body(gt_ref, valid_ref, logits_ref, ce_ref):
    b = pl.program_id(0)
    i = pl.program_id(1)

    @pl.when((b == 0) & (i == 0))
    def _init():
        ce_ref[0] = 0.0

    gt = gt_ref[0]
    vmask = valid_ref[0]
    mask = jnp.where(gt < MAXD, vmask, 0.0)
    labels = jnp.clip(gt, 0.0, 381.0) / INTERVAL

    m = jnp.max(logits_ref[0], axis=0)

    CH = 8
    C = 128
    s = jnp.zeros(m.shape, jnp.float32)
    g = jnp.zeros(m.shape, jnp.float32)
    for j in range(0, C, CH):
        xc = logits_ref[0, j:j + CH]
        s = s + jnp.sum(jnp.exp(xc - m[None]), axis=0)
        cf = (lax.broadcasted_iota(jnp.int32, (CH, 1, 1), 0) + j).astype(jnp.float32)
        w = jnp.maximum(1.0 - jnp.abs(labels[None] - cf), 0.0)
        g = g + jnp.sum(xc * w, axis=0)
    lse = m + jnp.log(s)

    ce_ref[0] += jnp.sum((lse - g) * mask)


def _tc_ce_sum(gt, validf, logits):
    B, C, H, W = logits.shape
    nb = H // BH
    (ce,) = pl.pallas_call(
        _tc_body,
        grid=(B, nb),
        in_specs=[
            pl.BlockSpec((1, BH, W), lambda b, i: (b, i, 0)),
            pl.BlockSpec((1, BH, W), lambda b, i: (b, i, 0)),
            pl.BlockSpec((1, C, BH, W), lambda b, i: (b, 0, i, 0)),
        ],
        out_specs=[pl.BlockSpec(memory_space=pltpu.SMEM)],
        out_shape=[jax.ShapeDtypeStruct((1,), jnp.float32)],
    )(gt, validf, logits)
    return ce[0]


def _make_sc_stats(n_pix):
    ppw = n_pix // NW
    nvec = ppw // LANES
    mesh = plsc.VectorSubcoreMesh(core_axis_name="c", subcore_axis_name="s")

    @functools.partial(
        pl.kernel,
        mesh=mesh,
        out_type=jax.ShapeDtypeStruct((NW, 2, LANES), jnp.float32),
        scratch_types=[
            pltpu.VMEM((ppw,), jnp.float32),
            pltpu.VMEM((ppw,), jnp.float32),
            pltpu.VMEM((ppw,), jnp.float32),
            pltpu.VMEM((2, LANES), jnp.float32),
        ],
    )
    def sc_stats(gt_hbm, pred_hbm, valid_hbm, out_hbm, gt_v, pred_v, valid_v, acc_v):
        wid = lax.axis_index("s") * NC + lax.axis_index("c")
        base = wid * ppw
        pltpu.sync_copy(gt_hbm.at[pl.ds(base, ppw)], gt_v)
        pltpu.sync_copy(pred_hbm.at[pl.ds(base, ppw)], pred_v)
        pltpu.sync_copy(valid_hbm.at[pl.ds(base, ppw)], valid_v)

        def body(i, carry):
            l1a, cnta = carry
            sl = pl.ds(i * LANES, LANES)
            gt = gt_v[sl]
            mask = jnp.where(gt < MAXD, valid_v[sl], 0.0)
            l1a = l1a + mask * jnp.abs(pred_v[sl] - gt)
            cnta = cnta + mask
            return l1a, cnta

        zero = jnp.zeros((LANES,), jnp.float32)
        l1a, cnta = lax.fori_loop(0, nvec, body, (zero, zero))
        acc_v[0] = l1a
        acc_v[1] = cnta
        pltpu.sync_copy(acc_v, out_hbm.at[wid])

    return sc_stats


def kernel(pred_disp, disp_logits, gt_disp, valid):
    B, C, H, W = disp_logits.shape
    pred_disp = pred_disp.astype(jnp.float32)
    gt_disp = gt_disp.astype(jnp.float32)
    validf = valid.astype(jnp.float32)
    logits = disp_logits.astype(jnp.float32)
    n_pix = B * H * W

    sc = _make_sc_stats(n_pix)
    parts = sc(
        gt_disp.reshape(n_pix),
        pred_disp.reshape(n_pix),
        validf.reshape(n_pix),
    )

    ce_sum = _tc_ce_sum(gt_disp, validf, logits)

    sums = jnp.sum(parts, axis=(0, 2))
    l1_sum, cnt = sums[0], sums[1]

    denom = cnt + 1e-6
    loss_disp = l1_sum / denom
    loss_logits = ce_sum / denom
    objective = WD * loss_disp + WL * loss_logits
    return objective, loss_disp, loss_logits


# speedup vs baseline: 1.0192x; 1.0192x over previous
import jax
import jax.numpy as jnp
from jax import lax
from jax.experimental import pallas as pl
from jax.experimental.pallas import tpu as pltpu

MAXD = 384.0
INTERVAL = 381.0 / 127.0
WD = 0.9
WL = 0.1

BH = 48


def _tc_body(pred_ref, gt_ref, valid_ref, logits_ref, l1_ref, ce_ref, cnt_ref):
    b = pl.program_id(0)
    i = pl.program_id(1)

    @pl.when((b == 0) & (i == 0))
    def _init():
        l1_ref[0] = 0.0
        ce_ref[0] = 0.0
        cnt_ref[0] = 0.0

    gt = gt_ref[0]
    pred = pred_ref[0]
    vmask = valid_ref[0]
    mask = jnp.where(gt < MAXD, vmask, 0.0)

    l1 = jnp.abs(pred - gt) * mask

    labels = jnp.clip(gt, 0.0, 381.0) / INTERVAL

    m = jnp.max(logits_ref[0], axis=0)

    CH = 8
    C = 128
    s = jnp.zeros(m.shape, jnp.float32)
    g = jnp.zeros(m.shape, jnp.float32)
    for j in range(0, C, CH):
        xc = logits_ref[0, j:j + CH]
        s = s + jnp.sum(jnp.exp(xc - m[None]), axis=0)
        cf = (lax.broadcasted_iota(jnp.int32, (CH, 1, 1), 0) + j).astype(jnp.float32)
        w = jnp.maximum(1.0 - jnp.abs(labels[None] - cf), 0.0)
        g = g + jnp.sum(xc * w, axis=0)
    lse = m + jnp.log(s)

    ce = (lse - g) * mask

    l1_ref[0] += jnp.sum(l1)
    ce_ref[0] += jnp.sum(ce)
    cnt_ref[0] += jnp.sum(mask)


def kernel(pred_disp, disp_logits, gt_disp, valid):
    B, C, H, W = disp_logits.shape
    pred_disp = pred_disp.astype(jnp.float32)
    gt_disp = gt_disp.astype(jnp.float32)
    validf = valid.astype(jnp.float32)
    logits = disp_logits.astype(jnp.float32)
    nb = H // BH

    l1_sum, ce_sum, cnt = pl.pallas_call(
        _tc_body,
        grid=(B, nb),
        in_specs=[
            pl.BlockSpec((1, BH, W), lambda b, i: (b, i, 0)),
            pl.BlockSpec((1, BH, W), lambda b, i: (b, i, 0)),
            pl.BlockSpec((1, BH, W), lambda b, i: (b, i, 0)),
            pl.BlockSpec((1, C, BH, W), lambda b, i: (b, 0, i, 0)),
        ],
        out_specs=[
            pl.BlockSpec(memory_space=pltpu.SMEM),
            pl.BlockSpec(memory_space=pltpu.SMEM),
            pl.BlockSpec(memory_space=pltpu.SMEM),
        ],
        out_shape=[jax.ShapeDtypeStruct((1,), jnp.float32)] * 3,
    )(pred_disp, gt_disp, validf, logits)

    denom = cnt[0] + 1e-6
    loss_disp = l1_sum[0] / denom
    loss_logits = ce_sum[0] / denom
    objective = WD * loss_disp + WL * loss_logits
    return objective, loss_disp, loss_logits


# speedup vs baseline: 1.0351x; 1.0155x over previous
import jax
import jax.numpy as jnp
from jax import lax
from jax.experimental import pallas as pl
from jax.experimental.pallas import tpu as pltpu

MAXD = 384.0
INTERVAL = 381.0 / 127.0
WD = 0.9
WL = 0.1

BH = 32


def _tc_body(pred_ref, gt_ref, valid_ref, logits_ref, l1_ref, ce_ref, cnt_ref):
    b = pl.program_id(0)
    i = pl.program_id(1)

    @pl.when((b == 0) & (i == 0))
    def _init():
        l1_ref[0] = 0.0
        ce_ref[0] = 0.0
        cnt_ref[0] = 0.0

    gt = gt_ref[0]
    pred = pred_ref[0]
    vmask = valid_ref[0]
    mask = jnp.where(gt < MAXD, vmask, 0.0)

    l1 = jnp.abs(pred - gt) * mask

    labels = jnp.clip(gt, 0.0, 381.0) / INTERVAL

    m = jnp.max(logits_ref[0], axis=0)

    CH = 16
    C = 128
    s = jnp.zeros(m.shape, jnp.float32)
    g = jnp.zeros(m.shape, jnp.float32)
    for j in range(0, C, CH):
        xc = logits_ref[0, j:j + CH]
        s = s + jnp.sum(jnp.exp(xc - m[None]), axis=0)
        cf = (lax.broadcasted_iota(jnp.int32, (CH, 1, 1), 0) + j).astype(jnp.float32)
        w = jnp.maximum(1.0 - jnp.abs(labels[None] - cf), 0.0)
        g = g + jnp.sum(xc * w, axis=0)
    lse = m + jnp.log(s)

    ce = (lse - g) * mask

    l1_ref[0] += jnp.sum(l1)
    ce_ref[0] += jnp.sum(ce)
    cnt_ref[0] += jnp.sum(mask)


def kernel(pred_disp, disp_logits, gt_disp, valid):
    B, C, H, W = disp_logits.shape
    pred_disp = pred_disp.astype(jnp.float32)
    gt_disp = gt_disp.astype(jnp.float32)
    validf = valid.astype(jnp.float32)
    logits = disp_logits.astype(jnp.float32)
    nb = H // BH

    l1_sum, ce_sum, cnt = pl.pallas_call(
        _tc_body,
        grid=(B, nb),
        in_specs=[
            pl.BlockSpec((1, BH, W), lambda b, i: (b, i, 0)),
            pl.BlockSpec((1, BH, W), lambda b, i: (b, i, 0)),
            pl.BlockSpec((1, BH, W), lambda b, i: (b, i, 0)),
            pl.BlockSpec((1, C, BH, W), lambda b, i: (b, 0, i, 0)),
        ],
        out_specs=[
            pl.BlockSpec(memory_space=pltpu.SMEM),
            pl.BlockSpec(memory_space=pltpu.SMEM),
            pl.BlockSpec(memory_space=pltpu.SMEM),
        ],
        out_shape=[jax.ShapeDtypeStruct((1,), jnp.float32)] * 3,
    )(pred_disp, gt_disp, validf, logits)

    denom = cnt[0] + 1e-6
    loss_disp = l1_sum[0] / denom
    loss_logits = ce_sum[0] / denom
    objective = WD * loss_disp + WL * loss_logits
    return objective, loss_disp, loss_logits


# speedup vs baseline: 1.2337x; 1.1919x over previous
import jax
import jax.numpy as jnp
from jax import lax
from jax.experimental import pallas as pl
from jax.experimental.pallas import tpu as pltpu

MAXD = 384.0
INTERVAL = 381.0 / 127.0
WD = 0.9
WL = 0.1

BH = 32


def _tc_body(pred_ref, gt_ref, valid_ref, logits_ref, l1_ref, ce_ref, cnt_ref):
    b = pl.program_id(0)
    i = pl.program_id(1)

    @pl.when((b == 0) & (i == 0))
    def _init():
        l1_ref[0] = 0.0
        ce_ref[0] = 0.0
        cnt_ref[0] = 0.0

    gt = gt_ref[0]
    pred = pred_ref[0]
    vmask = valid_ref[0]
    mask = jnp.where(gt < MAXD, vmask, 0.0)

    l1 = jnp.abs(pred - gt) * mask

    labels = jnp.clip(gt, 0.0, 381.0) / INTERVAL

    m = jnp.max(logits_ref[0], axis=0)

    CH = 4
    C = 128
    s = jnp.zeros(m.shape, jnp.float32)
    g = jnp.zeros(m.shape, jnp.float32)
    for j in range(0, C, CH):
        xc = logits_ref[0, j:j + CH]
        s = s + jnp.sum(jnp.exp(xc - m[None]), axis=0)
        cf = (lax.broadcasted_iota(jnp.int32, (CH, 1, 1), 0) + j).astype(jnp.float32)
        w = jnp.maximum(1.0 - jnp.abs(labels[None] - cf), 0.0)
        g = g + jnp.sum(xc * w, axis=0)
    lse = m + jnp.log(s)

    ce = (lse - g) * mask

    l1_ref[0] += jnp.sum(l1)
    ce_ref[0] += jnp.sum(ce)
    cnt_ref[0] += jnp.sum(mask)


def kernel(pred_disp, disp_logits, gt_disp, valid):
    B, C, H, W = disp_logits.shape
    pred_disp = pred_disp.astype(jnp.float32)
    gt_disp = gt_disp.astype(jnp.float32)
    validf = valid.astype(jnp.float32)
    logits = disp_logits.astype(jnp.float32)
    nb = H // BH

    l1_sum, ce_sum, cnt = pl.pallas_call(
        _tc_body,
        grid=(B, nb),
        in_specs=[
            pl.BlockSpec((1, BH, W), lambda b, i: (b, i, 0)),
            pl.BlockSpec((1, BH, W), lambda b, i: (b, i, 0)),
            pl.BlockSpec((1, BH, W), lambda b, i: (b, i, 0)),
            pl.BlockSpec((1, C, BH, W), lambda b, i: (b, 0, i, 0)),
        ],
        out_specs=[
            pl.BlockSpec(memory_space=pltpu.SMEM),
            pl.BlockSpec(memory_space=pltpu.SMEM),
            pl.BlockSpec(memory_space=pltpu.SMEM),
        ],
        out_shape=[jax.ShapeDtypeStruct((1,), jnp.float32)] * 3,
    )(pred_disp, gt_disp, validf, logits)

    denom = cnt[0] + 1e-6
    loss_disp = l1_sum[0] / denom
    loss_logits = ce_sum[0] / denom
    objective = WD * loss_disp + WL * loss_logits
    return objective, loss_disp, loss_logits


# speedup vs baseline: 1.3321x; 1.0798x over previous
import jax
import jax.numpy as jnp
from jax import lax
from jax.experimental import pallas as pl
from jax.experimental.pallas import tpu as pltpu

MAXD = 384.0
INTERVAL = 381.0 / 127.0
WD = 0.9
WL = 0.1

BH = 32


def _tc_body(pred_ref, gt_ref, valid_ref, logits_ref, l1_ref, ce_ref, cnt_ref):
    b = pl.program_id(0)
    i = pl.program_id(1)

    @pl.when((b == 0) & (i == 0))
    def _init():
        l1_ref[0] = 0.0
        ce_ref[0] = 0.0
        cnt_ref[0] = 0.0

    gt = gt_ref[0]
    pred = pred_ref[0]
    vmask = valid_ref[0]
    mask = jnp.where(gt < MAXD, vmask, 0.0)

    l1 = jnp.abs(pred - gt) * mask

    labels = jnp.clip(gt, 0.0, 381.0) / INTERVAL

    m = jnp.max(logits_ref[0], axis=0)

    CH = 2
    C = 128
    s = jnp.zeros(m.shape, jnp.float32)
    g = jnp.zeros(m.shape, jnp.float32)
    for j in range(0, C, CH):
        xc = logits_ref[0, j:j + CH]
        s = s + jnp.sum(jnp.exp(xc - m[None]), axis=0)
        cf = (lax.broadcasted_iota(jnp.int32, (CH, 1, 1), 0) + j).astype(jnp.float32)
        w = jnp.maximum(1.0 - jnp.abs(labels[None] - cf), 0.0)
        g = g + jnp.sum(xc * w, axis=0)
    lse = m + jnp.log(s)

    ce = (lse - g) * mask

    l1_ref[0] += jnp.sum(l1)
    ce_ref[0] += jnp.sum(ce)
    cnt_ref[0] += jnp.sum(mask)


def kernel(pred_disp, disp_logits, gt_disp, valid):
    B, C, H, W = disp_logits.shape
    pred_disp = pred_disp.astype(jnp.float32)
    gt_disp = gt_disp.astype(jnp.float32)
    validf = valid.astype(jnp.float32)
    logits = disp_logits.astype(jnp.float32)
    nb = H // BH

    l1_sum, ce_sum, cnt = pl.pallas_call(
        _tc_body,
        grid=(B, nb),
        in_specs=[
            pl.BlockSpec((1, BH, W), lambda b, i: (b, i, 0)),
            pl.BlockSpec((1, BH, W), lambda b, i: (b, i, 0)),
            pl.BlockSpec((1, BH, W), lambda b, i: (b, i, 0)),
            pl.BlockSpec((1, C, BH, W), lambda b, i: (b, 0, i, 0)),
        ],
        out_specs=[
            pl.BlockSpec(memory_space=pltpu.SMEM),
            pl.BlockSpec(memory_space=pltpu.SMEM),
            pl.BlockSpec(memory_space=pltpu.SMEM),
        ],
        out_shape=[jax.ShapeDtypeStruct((1,), jnp.float32)] * 3,
    )(pred_disp, gt_disp, validf, logits)

    denom = cnt[0] + 1e-6
    loss_disp = l1_sum[0] / denom
    loss_logits = ce_sum[0] / denom
    objective = WD * loss_disp + WL * loss_logits
    return objective, loss_disp, loss_logits


# speedup vs baseline: 1.3569x; 1.0186x over previous
import jax
import jax.numpy as jnp
from jax import lax
from jax.experimental import pallas as pl
from jax.experimental.pallas import tpu as pltpu

MAXD = 384.0
INTERVAL = 381.0 / 127.0
WD = 0.9
WL = 0.1

BH = 32


def _tc_body(pred_ref, gt_ref, valid_ref, logits_ref, l1_ref, ce_ref, cnt_ref):
    b = pl.program_id(0)
    i = pl.program_id(1)

    @pl.when((b == 0) & (i == 0))
    def _init():
        l1_ref[0] = 0.0
        ce_ref[0] = 0.0
        cnt_ref[0] = 0.0

    gt = gt_ref[0]
    pred = pred_ref[0]
    vmask = valid_ref[0]
    mask = jnp.where(gt < MAXD, vmask, 0.0)

    l1 = jnp.abs(pred - gt) * mask

    labels = jnp.clip(gt, 0.0, 381.0) / INTERVAL

    m = jnp.max(logits_ref[0], axis=0)

    CH = 1
    C = 128
    s = jnp.zeros(m.shape, jnp.float32)
    g = jnp.zeros(m.shape, jnp.float32)
    for j in range(0, C, CH):
        xc = logits_ref[0, j:j + CH]
        s = s + jnp.sum(jnp.exp(xc - m[None]), axis=0)
        cf = (lax.broadcasted_iota(jnp.int32, (CH, 1, 1), 0) + j).astype(jnp.float32)
        w = jnp.maximum(1.0 - jnp.abs(labels[None] - cf), 0.0)
        g = g + jnp.sum(xc * w, axis=0)
    lse = m + jnp.log(s)

    ce = (lse - g) * mask

    l1_ref[0] += jnp.sum(l1)
    ce_ref[0] += jnp.sum(ce)
    cnt_ref[0] += jnp.sum(mask)


def kernel(pred_disp, disp_logits, gt_disp, valid):
    B, C, H, W = disp_logits.shape
    pred_disp = pred_disp.astype(jnp.float32)
    gt_disp = gt_disp.astype(jnp.float32)
    validf = valid.astype(jnp.float32)
    logits = disp_logits.astype(jnp.float32)
    nb = H // BH

    l1_sum, ce_sum, cnt = pl.pallas_call(
        _tc_body,
        grid=(B, nb),
        in_specs=[
            pl.BlockSpec((1, BH, W), lambda b, i: (b, i, 0)),
            pl.BlockSpec((1, BH, W), lambda b, i: (b, i, 0)),
            pl.BlockSpec((1, BH, W), lambda b, i: (b, i, 0)),
            pl.BlockSpec((1, C, BH, W), lambda b, i: (b, 0, i, 0)),
        ],
        out_specs=[
            pl.BlockSpec(memory_space=pltpu.SMEM),
            pl.BlockSpec(memory_space=pltpu.SMEM),
            pl.BlockSpec(memory_space=pltpu.SMEM),
        ],
        out_shape=[jax.ShapeDtypeStruct((1,), jnp.float32)] * 3,
    )(pred_disp, gt_disp, validf, logits)

    denom = cnt[0] + 1e-6
    loss_disp = l1_sum[0] / denom
    loss_logits = ce_sum[0] / denom
    objective = WD * loss_disp + WL * loss_logits
    return objective, loss_disp, loss_logits
